# baseline (speedup 1.0000x reference)
import functools

import jax
import jax.numpy as jnp
from jax import lax
from jax.experimental import pallas as pl
from jax.experimental.pallas import tpu as pltpu
from jax.experimental.pallas import tpu_sc as plsc

L = 16


def _softplus(z):
    a = jnp.abs(z)
    t = jnp.exp(-a)
    w = t / (t + 2.0)
    w2 = w * w
    series = 1.0 + w2 * (
        (1.0 / 3.0) + w2 * ((1.0 / 5.0) + w2 * ((1.0 / 7.0) + w2 * (1.0 / 9.0)))
    )
    return jnp.maximum(z, 0.0) + 2.0 * w * series


@functools.partial(jax.jit, static_argnums=(4, 5, 6))
def _hier_softmax_sc(center, context, embeddings, probs_tensor, num_nodes, nlev, d):
    batch = center.shape[0]
    info = plsc.get_sparse_core_info()
    nw = info.num_cores * info.num_subcores
    c_per = batch // nw
    sub = 128
    nsub = c_per // sub
    groups = sub // L

    mesh = plsc.VectorSubcoreMesh(core_axis_name="c", subcore_axis_name="s")

    @functools.partial(
        pl.kernel,
        mesh=mesh,
        out_type=jax.ShapeDtypeStruct((batch,), jnp.float32),
        compiler_params=pltpu.CompilerParams(
            needs_layout_passes=False, use_tc_tiling_on_sc=False),
        scratch_types=[
            pltpu.VMEM((sub,), jnp.int32),
            pltpu.VMEM((sub,), jnp.int32),
            pltpu.VMEM((nlev, sub), jnp.int32),
            pltpu.VMEM((sub, d), jnp.float32),
            pltpu.VMEM((nlev * sub, d), jnp.float32),
            pltpu.VMEM((sub,), jnp.float32),
            pltpu.SemaphoreType.DMA,
            pltpu.SemaphoreType.DMA,
        ],
    )
    def body(center_hbm, context_hbm, emb_hbm, probs_hbm, out_hbm,
             cen_v, ctx_v, pidx_v, emb_buf, probs_buf, out_v, sem_e, sem_p):
        wid = lax.axis_index("s") * info.num_cores + lax.axis_index("c")
        lane = lax.iota(jnp.int32, L)

        for s in range(nsub):
            base = wid * c_per + s * sub
            pltpu.sync_copy(center_hbm.at[pl.ds(base, sub)], cen_v)
            pltpu.sync_copy(context_hbm.at[pl.ds(base, sub)], ctx_v)
            emb_dma = pltpu.async_copy(emb_hbm.at[cen_v], emb_buf, sem_e)

            def build_idx(g, _):
                node = ctx_v[pl.ds(g * L, L)] + num_nodes

                def per_level(i, _):
                    pidx_v[i - 1, pl.ds(g * L, L)] = node >> i
                    return 0

                lax.fori_loop(1, nlev + 1, per_level, 0)
                return 0

            lax.fori_loop(0, groups, build_idx, 0)

            probs_dmas = [
                pltpu.async_copy(
                    probs_hbm.at[pidx_v.at[i]],
                    probs_buf.at[pl.ds(i * sub, sub)],
                    sem_p,
                )
                for i in range(nlev)
            ]
            emb_dma.wait()
            for dma in probs_dmas:
                dma.wait()

            def per_group(g, _):
                rvec = g * L + lane
                e = [
                    plsc.load_gather(emb_buf, [rvec, jnp.full((L,), dd, jnp.int32)])
                    for dd in range(d)
                ]
                node = ctx_v[pl.ds(g * L, L)] + num_nodes

                def per_level(i, out_acc):
                    prvec = (i - 1) * sub + rvec
                    acc0 = jnp.zeros((L,), jnp.float32)
                    acc1 = jnp.zeros((L,), jnp.float32)
                    acc2 = jnp.zeros((L,), jnp.float32)
                    acc3 = jnp.zeros((L,), jnp.float32)
                    for dd in range(0, d, 4):
                        acc0 += plsc.load_gather(
                            probs_buf, [prvec, jnp.full((L,), dd, jnp.int32)]) * e[dd]
                        acc1 += plsc.load_gather(
                            probs_buf, [prvec, jnp.full((L,), dd + 1, jnp.int32)]) * e[dd + 1]
                        acc2 += plsc.load_gather(
                            probs_buf, [prvec, jnp.full((L,), dd + 2, jnp.int32)]) * e[dd + 2]
                        acc3 += plsc.load_gather(
                            probs_buf, [prvec, jnp.full((L,), dd + 3, jnp.int32)]) * e[dd + 3]
                    dot = (acc0 + acc1) + (acc2 + acc3)
                    nid = node >> i
                    sgn = jnp.where((nid & 1) == 0, 1.0, -1.0).astype(jnp.float32)
                    return out_acc + _softplus(-sgn * dot)

                out_acc = lax.fori_loop(
                    1, nlev + 1, per_level, jnp.zeros((L,), jnp.float32))
                out_v[pl.ds(g * L, L)] = out_acc
                return 0

            lax.fori_loop(0, groups, per_group, 0)
            pltpu.sync_copy(out_v, out_hbm.at[pl.ds(base, sub)])

    return body(center, context, embeddings, probs_tensor)


def kernel(center, context, embeddings, probs_tensor):
    num_nodes = embeddings.shape[0]
    nlev = num_nodes.bit_length() - 2
    out = _hier_softmax_sc(
        center, context, embeddings, probs_tensor,
        num_nodes, nlev, embeddings.shape[1])
    return out.reshape(-1, 1)


# speedup vs baseline: 5.0572x; 5.0572x over previous
import functools

import jax
import jax.numpy as jnp
from jax import lax
from jax.experimental import pallas as pl
from jax.experimental.pallas import tpu as pltpu
from jax.experimental.pallas import tpu_sc as plsc

L = 16
D = 32
SUB = 32
NSHAL = 10
SLAB_ROWS = 2048


def _softplus(z):
    t = jnp.exp(-jnp.abs(z))
    w = t / (t + 2.0)
    w2 = w * w
    series = 1.0 + w2 * (
        (1.0 / 3.0) + w2 * ((1.0 / 5.0) + w2 * ((1.0 / 7.0) + w2 * (1.0 / 9.0)))
    )
    return jnp.maximum(z, 0.0) + 2.0 * w * series


def _kp(d, n16):
    return (d >> 3) * n16 + (d & 7) * 8


@functools.partial(jax.jit, static_argnums=(4,))
def _hier_softmax_sc(center, context, emb_w, probs_w, num_nodes):
    batch = center.shape[0]
    info = plsc.get_sparse_core_info()
    nw = info.num_cores * info.num_subcores
    c_per = batch // nw
    nsub = c_per // SUB
    groups = SUB // L
    nlev = num_nodes.bit_length() - 2
    np16 = probs_w.shape[0] // 4
    ne16 = emb_w.shape[0] // 4
    slab_seg = SLAB_ROWS // 128 * 8 * 8

    mesh = plsc.VectorSubcoreMesh(core_axis_name="c", subcore_axis_name="s")

    @functools.partial(
        pl.kernel,
        mesh=mesh,
        out_type=jax.ShapeDtypeStruct((batch,), jnp.float32),
        compiler_params=pltpu.CompilerParams(
            needs_layout_passes=False, use_tc_tiling_on_sc=False),
        scratch_types=[
            pltpu.VMEM((c_per,), jnp.int32),
            pltpu.VMEM((c_per,), jnp.int32),
            pltpu.VMEM((4 * slab_seg, L), jnp.float32),
            pltpu.VMEM((SUB * D,), jnp.int32),
            pltpu.VMEM((SUB * D, L), jnp.float32),
            pltpu.VMEM((SUB * D,), jnp.int32),
            pltpu.VMEM((SUB * D,), jnp.int32),
            pltpu.VMEM((SUB * D, L), jnp.float32),
            pltpu.VMEM((SUB * D, L), jnp.float32),
            pltpu.VMEM((c_per,), jnp.float32),
            pltpu.SemaphoreType.DMA,
            pltpu.SemaphoreType.DMA,
            pltpu.SemaphoreType.DMA,
            pltpu.SemaphoreType.DMA,
        ],
    )
    def body(center_hbm, context_hbm, emb_hbm, probs_hbm, out_hbm,
             cen_v, ctx_v, slab_v, eidx_v, ebuf_v, pidx0_v, pidx1_v,
             pbuf0_v, pbuf1_v, out_v, sem_s, sem_e, sem_p, sem_q):
        wid = lax.axis_index("s") * info.num_cores + lax.axis_index("c")
        lane = lax.iota(jnp.int32, L)
        base = wid * c_per

        pltpu.sync_copy(center_hbm.at[pl.ds(base, c_per)], cen_v)
        pltpu.sync_copy(context_hbm.at[pl.ds(base, c_per)], ctx_v)
        slab_dmas = [
            pltpu.async_copy(
                probs_hbm.at[pl.ds(k * np16, slab_seg)],
                slab_v.at[pl.ds(k * slab_seg, slab_seg)],
                sem_s,
            )
            for k in range(4)
        ]
        for dma in slab_dmas:
            dma.wait()

        def view_base(r):
            return (r >> 7) * 64 + ((r & 127) >> 4)

        def build_idx(idx_ref, rvals):
            for g in range(groups):
                vb = view_base(rvals[g])
                for d in range(D):
                    q = d * SUB + g * L
                    idx_ref[pl.ds(q, L)] = vb + _kp(d, np16)

        def sub_body(s, _):
            sb = s * SUB
            cen = [cen_v[pl.ds(sb + g * L, L)] for g in range(groups)]
            node = [ctx_v[pl.ds(sb + g * L, L)] + num_nodes
                    for g in range(groups)]

            for g in range(groups):
                vb = view_base(cen[g])
                for d in range(D):
                    q = d * SUB + g * L
                    eidx_v[pl.ds(q, L)] = vb + _kp(d, ne16)
            e_dma = pltpu.async_copy(emb_hbm.at[eidx_v], ebuf_v, sem_e)

            elane_l = [c & 15 for c in cen]

            build_idx(pidx0_v, [n >> 1 for n in node])
            pltpu.async_copy(probs_hbm.at[pidx0_v], pbuf0_v, sem_p)
            build_idx(pidx1_v, [n >> 2 for n in node])
            pltpu.async_copy(probs_hbm.at[pidx1_v], pbuf1_v, sem_q)
            e_dma.wait()

            def level_term(buf_ref, i, g, r, row_of_d):
                dot = dot_group_local(buf_ref, g, r, row_of_d)
                sgn = jnp.where((r & 1) == 0, 1.0, -1.0).astype(jnp.float32)
                return _softplus(-sgn * dot)

            def dot_group_local(buf_ref, g, r, row_of_d):
                lane_sel = r & 15
                accs = [jnp.zeros((L,), jnp.float32) for _ in range(4)]
                for d in range(D):
                    pv = plsc.load_gather(buf_ref, [row_of_d(d, g), lane_sel])
                    ev = plsc.load_gather(
                        ebuf_v, [d * SUB + g * L + lane, elane_l[g]])
                    accs[d % 4] = accs[d % 4] + pv * ev
                return (accs[0] + accs[1]) + (accs[2] + accs[3])

            def staged_row(d, g):
                return jnp.full((L,), d * SUB + g * L, jnp.int32) + lane

            def shallow_pair(t, acc):
                i = 1 + 2 * t
                acc0, acc1 = acc
                pltpu.make_async_copy(
                    probs_hbm.at[pidx0_v], pbuf0_v, sem_p).wait()
                acc0 = acc0 + level_term(pbuf0_v, i, 0, node[0] >> i, staged_row)
                acc1 = acc1 + level_term(pbuf0_v, i, 1, node[1] >> i, staged_row)

                @pl.when(i + 2 <= NSHAL)
                def _():
                    build_idx(pidx0_v, [n >> (i + 2) for n in node])
                    pltpu.async_copy(probs_hbm.at[pidx0_v], pbuf0_v, sem_p)

                pltpu.make_async_copy(
                    probs_hbm.at[pidx1_v], pbuf1_v, sem_q).wait()
                acc0 = acc0 + level_term(
                    pbuf1_v, i + 1, 0, node[0] >> (i + 1), staged_row)
                acc1 = acc1 + level_term(
                    pbuf1_v, i + 1, 1, node[1] >> (i + 1), staged_row)

                @pl.when(i + 3 <= NSHAL)
                def _():
                    build_idx(pidx1_v, [n >> (i + 3) for n in node])
                    pltpu.async_copy(probs_hbm.at[pidx1_v], pbuf1_v, sem_q)

                return (acc0, acc1)

            acc = lax.fori_loop(
                0, NSHAL // 2, shallow_pair,
                (jnp.zeros((L,), jnp.float32), jnp.zeros((L,), jnp.float32)))

            def slab_row(i):
                def row_of_d(d, g):
                    r = node[g] >> i
                    return view_base(r) + _kp(d, slab_seg)
                return row_of_d

            def deep_level(i, acc):
                acc0, acc1 = acc
                acc0 = acc0 + level_term(slab_v, i, 0, node[0] >> i, slab_row(i))
                acc1 = acc1 + level_term(slab_v, i, 1, node[1] >> i, slab_row(i))
                return (acc0, acc1)

            acc = lax.fori_loop(NSHAL + 1, nlev + 1, deep_level, acc)

            out_v[pl.ds(sb, L)] = acc[0]
            out_v[pl.ds(sb + L, L)] = acc[1]
            return 0

        lax.fori_loop(0, nsub, sub_body, 0)
        pltpu.sync_copy(out_v, out_hbm.at[pl.ds(base, c_per)])

    return body(center, context, emb_w, probs_w)


def kernel(center, context, embeddings, probs_tensor):
    num_nodes = embeddings.shape[0]
    nr = probs_tensor.shape[0]
    probs_w = (
        probs_tensor.T.reshape(4, 8, nr // 128, 128)
        .transpose(0, 2, 1, 3)
        .reshape(nr * D // L, L)
    )
    emb_w = (
        embeddings.T.reshape(4, 8, num_nodes // 128, 128)
        .transpose(0, 2, 1, 3)
        .reshape(num_nodes * D // L, L)
    )
    out = _hier_softmax_sc(center, context, emb_w, probs_w, num_nodes)
    return out.reshape(-1, 1)


# speedup vs baseline: 6.1285x; 1.2119x over previous
import functools

import jax
import jax.numpy as jnp
from jax import lax
from jax.experimental import pallas as pl
from jax.experimental.pallas import tpu as pltpu
from jax.experimental.pallas import tpu_sc as plsc

L = 16
D = 32
SUB = 32
NSHAL = 10
SLAB_ROWS = 2048


def _softplus(z):
    t = jnp.exp(-jnp.abs(z))
    w = t / (t + 2.0)
    w2 = w * w
    series = 1.0 + w2 * (
        (1.0 / 3.0) + w2 * ((1.0 / 5.0) + w2 * ((1.0 / 7.0) + w2 * (1.0 / 9.0)))
    )
    return jnp.maximum(z, 0.0) + 2.0 * w * series


def _kp(d, n16):
    return (d >> 3) * n16 + (d & 7) * 8


@functools.partial(jax.jit, static_argnums=(4,))
def _hier_softmax_sc(center, context, emb_w, probs_w, num_nodes):
    batch = center.shape[0]
    info = plsc.get_sparse_core_info()
    nw = info.num_cores * info.num_subcores
    c_per = batch // nw
    nsub = c_per // SUB
    groups = SUB // L
    nlev = num_nodes.bit_length() - 2
    np16 = probs_w.shape[0] // 4
    ne16 = emb_w.shape[0] // 4
    slab_seg = SLAB_ROWS // 128 * 8 * 8

    mesh = plsc.VectorSubcoreMesh(core_axis_name="c", subcore_axis_name="s")

    @functools.partial(
        pl.kernel,
        mesh=mesh,
        out_type=jax.ShapeDtypeStruct((batch,), jnp.float32),
        compiler_params=pltpu.CompilerParams(
            needs_layout_passes=False, use_tc_tiling_on_sc=False),
        scratch_types=[
            pltpu.VMEM((c_per,), jnp.int32),
            pltpu.VMEM((c_per,), jnp.int32),
            pltpu.VMEM((4 * slab_seg, L), jnp.float32),
            pltpu.VMEM((SUB * D,), jnp.int32),
            pltpu.VMEM((SUB * D, L), jnp.float32),
            pltpu.VMEM((D, SUB), jnp.float32),
            pltpu.VMEM((SUB * D,), jnp.int32),
            pltpu.VMEM((SUB * D,), jnp.int32),
            pltpu.VMEM((SUB * D, L), jnp.float32),
            pltpu.VMEM((SUB * D, L), jnp.float32),
            pltpu.VMEM((c_per,), jnp.float32),
            pltpu.SemaphoreType.DMA,
            pltpu.SemaphoreType.DMA,
            pltpu.SemaphoreType.DMA,
            pltpu.SemaphoreType.DMA,
        ],
    )
    def body(center_hbm, context_hbm, emb_hbm, probs_hbm, out_hbm,
             cen_v, ctx_v, slab_v, eidx_v, ebuf_v, ecmp_v, pidx0_v, pidx1_v,
             pbuf0_v, pbuf1_v, out_v, sem_s, sem_e, sem_p, sem_q):
        wid = lax.axis_index("s") * info.num_cores + lax.axis_index("c")
        lane = lax.iota(jnp.int32, L)
        base = wid * c_per

        pltpu.sync_copy(center_hbm.at[pl.ds(base, c_per)], cen_v)
        pltpu.sync_copy(context_hbm.at[pl.ds(base, c_per)], ctx_v)
        slab_dmas = [
            pltpu.async_copy(
                probs_hbm.at[pl.ds(k * np16, slab_seg)],
                slab_v.at[pl.ds(k * slab_seg, slab_seg)],
                sem_s,
            )
            for k in range(4)
        ]
        for dma in slab_dmas:
            dma.wait()

        def view_base(r):
            return (r >> 7) * 64 + ((r & 127) >> 4)

        def build_idx(idx_ref, rvals):
            for g in range(groups):
                vb = view_base(rvals[g])
                for d in range(D):
                    q = d * SUB + g * L
                    idx_ref[pl.ds(q, L)] = vb + _kp(d, np16)

        def prefetch_sub(sb):
            cen = [cen_v[pl.ds(sb + g * L, L)] for g in range(groups)]
            node = [ctx_v[pl.ds(sb + g * L, L)] + num_nodes
                    for g in range(groups)]
            for g in range(groups):
                vb = view_base(cen[g])
                for d in range(D):
                    q = d * SUB + g * L
                    eidx_v[pl.ds(q, L)] = vb + _kp(d, ne16)
            pltpu.async_copy(emb_hbm.at[eidx_v], ebuf_v, sem_e)
            build_idx(pidx0_v, [n >> 1 for n in node])
            pltpu.async_copy(probs_hbm.at[pidx0_v], pbuf0_v, sem_p)
            build_idx(pidx1_v, [n >> 2 for n in node])
            pltpu.async_copy(probs_hbm.at[pidx1_v], pbuf1_v, sem_q)

        prefetch_sub(0)

        def sub_body(s, _):
            sb = s * SUB
            cen = [cen_v[pl.ds(sb + g * L, L)] for g in range(groups)]
            node = [ctx_v[pl.ds(sb + g * L, L)] + num_nodes
                    for g in range(groups)]
            elane_l = [c & 15 for c in cen]

            pltpu.make_async_copy(emb_hbm.at[eidx_v], ebuf_v, sem_e).wait()
            for g in range(groups):
                for d in range(D):
                    v = plsc.load_gather(
                        ebuf_v, [d * SUB + g * L + lane, elane_l[g]])
                    ecmp_v[d, pl.ds(g * L, L)] = v

            def level_term(buf_ref, i, g, r, row_of_d):
                dot = dot_group_local(buf_ref, g, r, row_of_d)
                sgn = jnp.where((r & 1) == 0, 1.0, -1.0).astype(jnp.float32)
                return _softplus(-sgn * dot)

            def dot_group_local(buf_ref, g, r, row_of_d):
                lane_sel = r & 15
                accs = [jnp.zeros((L,), jnp.float32) for _ in range(4)]
                for d in range(D):
                    pv = plsc.load_gather(buf_ref, [row_of_d(d, g), lane_sel])
                    ev = ecmp_v[d, pl.ds(g * L, L)]
                    accs[d % 4] = accs[d % 4] + pv * ev
                return (accs[0] + accs[1]) + (accs[2] + accs[3])

            def staged_row(d, g):
                return jnp.full((L,), d * SUB + g * L, jnp.int32) + lane

            def shallow_pair(t, acc):
                i = 1 + 2 * t
                acc0, acc1 = acc
                pltpu.make_async_copy(
                    probs_hbm.at[pidx0_v], pbuf0_v, sem_p).wait()
                acc0 = acc0 + level_term(pbuf0_v, i, 0, node[0] >> i, staged_row)
                acc1 = acc1 + level_term(pbuf0_v, i, 1, node[1] >> i, staged_row)

                @pl.when(i + 2 <= NSHAL)
                def _():
                    build_idx(pidx0_v, [n >> (i + 2) for n in node])
                    pltpu.async_copy(probs_hbm.at[pidx0_v], pbuf0_v, sem_p)

                pltpu.make_async_copy(
                    probs_hbm.at[pidx1_v], pbuf1_v, sem_q).wait()
                acc0 = acc0 + level_term(
                    pbuf1_v, i + 1, 0, node[0] >> (i + 1), staged_row)
                acc1 = acc1 + level_term(
                    pbuf1_v, i + 1, 1, node[1] >> (i + 1), staged_row)

                @pl.when(i + 3 <= NSHAL)
                def _():
                    build_idx(pidx1_v, [n >> (i + 3) for n in node])
                    pltpu.async_copy(probs_hbm.at[pidx1_v], pbuf1_v, sem_q)

                return (acc0, acc1)

            acc = lax.fori_loop(
                0, NSHAL // 2, shallow_pair,
                (jnp.zeros((L,), jnp.float32), jnp.zeros((L,), jnp.float32)))

            @pl.when(s + 1 < nsub)
            def _():
                prefetch_sub(sb + SUB)

            def slab_row(i):
                def row_of_d(d, g):
                    r = node[g] >> i
                    return view_base(r) + _kp(d, slab_seg)
                return row_of_d

            def deep_level(i, acc):
                acc0, acc1 = acc
                acc0 = acc0 + level_term(slab_v, i, 0, node[0] >> i, slab_row(i))
                acc1 = acc1 + level_term(slab_v, i, 1, node[1] >> i, slab_row(i))
                return (acc0, acc1)

            acc = lax.fori_loop(NSHAL + 1, nlev + 1, deep_level, acc)

            out_v[pl.ds(sb, L)] = acc[0]
            out_v[pl.ds(sb + L, L)] = acc[1]
            return 0

        lax.fori_loop(0, nsub, sub_body, 0)
        pltpu.sync_copy(out_v, out_hbm.at[pl.ds(base, c_per)])

    return body(center, context, emb_w, probs_w)


def kernel(center, context, embeddings, probs_tensor):
    num_nodes = embeddings.shape[0]
    nr = probs_tensor.shape[0]
    probs_w = (
        probs_tensor.T.reshape(4, 8, nr // 128, 128)
        .transpose(0, 2, 1, 3)
        .reshape(nr * D // L, L)
    )
    emb_w = (
        embeddings.T.reshape(4, 8, num_nodes // 128, 128)
        .transpose(0, 2, 1, 3)
        .reshape(num_nodes * D // L, L)
    )
    out = _hier_softmax_sc(center, context, emb_w, probs_w, num_nodes)
    return out.reshape(-1, 1)


# speedup vs baseline: 6.6657x; 1.0876x over previous
import functools

import jax
import jax.numpy as jnp
from jax import lax
from jax.experimental import pallas as pl
from jax.experimental.pallas import tpu as pltpu
from jax.experimental.pallas import tpu_sc as plsc

L = 16
D = 32
SUB = 32
NSHAL = 9
SLAB_ROWS = 2048


def _softplus(z):
    t = jnp.exp(-jnp.abs(z))
    w = t / (t + 2.0)
    w2 = w * w
    series = 1.0 + w2 * (
        (1.0 / 3.0) + w2 * ((1.0 / 5.0) + w2 * ((1.0 / 7.0) + w2 * (1.0 / 9.0)))
    )
    return jnp.maximum(z, 0.0) + 2.0 * w * series


def _kp(d, n16):
    return (d >> 3) * n16 + (d & 7) * 8


@functools.partial(jax.jit, static_argnums=(4,))
def _hier_softmax_sc(center, context, emb_w, probs_w, num_nodes):
    batch = center.shape[0]
    info = plsc.get_sparse_core_info()
    nw = info.num_cores * info.num_subcores
    c_per = batch // nw
    nsub = c_per // SUB
    groups = SUB // L
    nlev = num_nodes.bit_length() - 2
    np16 = probs_w.shape[0] // 4
    ne16 = emb_w.shape[0] // 4
    slab_seg = SLAB_ROWS // 128 * 8 * 8

    mesh = plsc.VectorSubcoreMesh(core_axis_name="c", subcore_axis_name="s")

    @functools.partial(
        pl.kernel,
        mesh=mesh,
        out_type=jax.ShapeDtypeStruct((batch,), jnp.float32),
        compiler_params=pltpu.CompilerParams(
            needs_layout_passes=False, use_tc_tiling_on_sc=False),
        scratch_types=[
            pltpu.VMEM((c_per,), jnp.int32),
            pltpu.VMEM((c_per,), jnp.int32),
            pltpu.VMEM((4 * slab_seg, L), jnp.float32),
            pltpu.VMEM((SUB * D,), jnp.int32),
            pltpu.VMEM((SUB * D, L), jnp.float32),
            pltpu.VMEM((D, SUB), jnp.float32),
            pltpu.VMEM((SUB * D,), jnp.int32),
            pltpu.VMEM((SUB * D,), jnp.int32),
            pltpu.VMEM((SUB * D, L), jnp.float32),
            pltpu.VMEM((SUB * D, L), jnp.float32),
            pltpu.VMEM((c_per,), jnp.float32),
            pltpu.SemaphoreType.DMA,
            pltpu.SemaphoreType.DMA,
            pltpu.SemaphoreType.DMA,
            pltpu.SemaphoreType.DMA,
        ],
    )
    def body(center_hbm, context_hbm, emb_hbm, probs_hbm, out_hbm,
             cen_v, ctx_v, slab_v, eidx_v, ebuf_v, ecmp_v, pidx0_v, pidx1_v,
             pbuf0_v, pbuf1_v, out_v, sem_s, sem_e, sem_p, sem_q):
        wid = lax.axis_index("s") * info.num_cores + lax.axis_index("c")
        lane = lax.iota(jnp.int32, L)
        base = wid * c_per

        pltpu.sync_copy(center_hbm.at[pl.ds(base, c_per)], cen_v)
        pltpu.sync_copy(context_hbm.at[pl.ds(base, c_per)], ctx_v)
        slab_dmas = [
            pltpu.async_copy(
                probs_hbm.at[pl.ds(k * np16, slab_seg)],
                slab_v.at[pl.ds(k * slab_seg, slab_seg)],
                sem_s,
            )
            for k in range(4)
        ]
        for dma in slab_dmas:
            dma.wait()

        def view_base(r):
            return (r >> 7) * 64 + ((r & 127) >> 4)

        def build_idx(idx_ref, rvals):
            for g in range(groups):
                vb = view_base(rvals[g])
                for d in range(D):
                    q = d * SUB + g * L
                    idx_ref[pl.ds(q, L)] = vb + _kp(d, np16)

        def prefetch_sub(sb):
            cen = [cen_v[pl.ds(sb + g * L, L)] for g in range(groups)]
            node = [ctx_v[pl.ds(sb + g * L, L)] + num_nodes
                    for g in range(groups)]
            for g in range(groups):
                vb = view_base(cen[g])
                for d in range(D):
                    q = d * SUB + g * L
                    eidx_v[pl.ds(q, L)] = vb + _kp(d, ne16)
            pltpu.async_copy(emb_hbm.at[eidx_v], ebuf_v, sem_e)
            build_idx(pidx0_v, [n >> 1 for n in node])
            pltpu.async_copy(probs_hbm.at[pidx0_v], pbuf0_v, sem_p)
            build_idx(pidx1_v, [n >> 2 for n in node])
            pltpu.async_copy(probs_hbm.at[pidx1_v], pbuf1_v, sem_q)

        prefetch_sub(0)

        def sub_body(s, _):
            sb = s * SUB
            cen = [cen_v[pl.ds(sb + g * L, L)] for g in range(groups)]
            node = [ctx_v[pl.ds(sb + g * L, L)] + num_nodes
                    for g in range(groups)]
            elane_l = [c & 15 for c in cen]

            pltpu.make_async_copy(emb_hbm.at[eidx_v], ebuf_v, sem_e).wait()
            for g in range(groups):
                for d in range(D):
                    v = plsc.load_gather(
                        ebuf_v, [d * SUB + g * L + lane, elane_l[g]])
                    ecmp_v[d, pl.ds(g * L, L)] = v

            def level_term(buf_ref, i, g, r, row_of_d):
                dot = dot_group_local(buf_ref, g, r, row_of_d)
                sgn = jnp.where((r & 1) == 0, 1.0, -1.0).astype(jnp.float32)
                return _softplus(-sgn * dot)

            def dot_group_local(buf_ref, g, r, row_of_d):
                lane_sel = r & 15
                accs = [jnp.zeros((L,), jnp.float32) for _ in range(4)]
                for d in range(D):
                    pv = plsc.load_gather(buf_ref, [row_of_d(d, g), lane_sel])
                    ev = ecmp_v[d, pl.ds(g * L, L)]
                    accs[d % 4] = accs[d % 4] + pv * ev
                return (accs[0] + accs[1]) + (accs[2] + accs[3])

            def staged_row(d, g):
                return jnp.full((L,), d * SUB + g * L, jnp.int32) + lane

            def shallow_pair(t, acc):
                i = 1 + 2 * t
                acc0, acc1 = acc
                pltpu.make_async_copy(
                    probs_hbm.at[pidx0_v], pbuf0_v, sem_p).wait()
                acc0 = acc0 + level_term(pbuf0_v, i, 0, node[0] >> i, staged_row)
                acc1 = acc1 + level_term(pbuf0_v, i, 1, node[1] >> i, staged_row)

                @pl.when(i + 2 <= NSHAL)
                def _():
                    build_idx(pidx0_v, [n >> (i + 2) for n in node])
                    pltpu.async_copy(probs_hbm.at[pidx0_v], pbuf0_v, sem_p)

                pltpu.make_async_copy(
                    probs_hbm.at[pidx1_v], pbuf1_v, sem_q).wait()
                acc0 = acc0 + level_term(
                    pbuf1_v, i + 1, 0, node[0] >> (i + 1), staged_row)
                acc1 = acc1 + level_term(
                    pbuf1_v, i + 1, 1, node[1] >> (i + 1), staged_row)

                @pl.when(i + 3 <= NSHAL)
                def _():
                    build_idx(pidx1_v, [n >> (i + 3) for n in node])
                    pltpu.async_copy(probs_hbm.at[pidx1_v], pbuf1_v, sem_q)

                return (acc0, acc1)

            acc = lax.fori_loop(
                0, NSHAL // 2, shallow_pair,
                (jnp.zeros((L,), jnp.float32), jnp.zeros((L,), jnp.float32)))

            pltpu.make_async_copy(
                probs_hbm.at[pidx0_v], pbuf0_v, sem_p).wait()
            acc = (
                acc[0] + level_term(
                    pbuf0_v, NSHAL, 0, node[0] >> NSHAL, staged_row),
                acc[1] + level_term(
                    pbuf0_v, NSHAL, 1, node[1] >> NSHAL, staged_row),
            )

            @pl.when(s + 1 < nsub)
            def _():
                prefetch_sub(sb + SUB)

            def slab_row(i):
                def row_of_d(d, g):
                    r = node[g] >> i
                    return view_base(r) + _kp(d, slab_seg)
                return row_of_d

            def deep_level(i, acc):
                acc0, acc1 = acc
                acc0 = acc0 + level_term(slab_v, i, 0, node[0] >> i, slab_row(i))
                acc1 = acc1 + level_term(slab_v, i, 1, node[1] >> i, slab_row(i))
                return (acc0, acc1)

            acc = lax.fori_loop(NSHAL + 1, nlev + 1, deep_level, acc)

            out_v[pl.ds(sb, L)] = acc[0]
            out_v[pl.ds(sb + L, L)] = acc[1]
            return 0

        lax.fori_loop(0, nsub, sub_body, 0)
        pltpu.sync_copy(out_v, out_hbm.at[pl.ds(base, c_per)])

    return body(center, context, emb_w, probs_w)


def kernel(center, context, embeddings, probs_tensor):
    num_nodes = embeddings.shape[0]
    nr = probs_tensor.shape[0]
    probs_w = (
        probs_tensor.T.reshape(4, 8, nr // 128, 128)
        .transpose(0, 2, 1, 3)
        .reshape(nr * D // L, L)
    )
    emb_w = (
        embeddings.T.reshape(4, 8, num_nodes // 128, 128)
        .transpose(0, 2, 1, 3)
        .reshape(num_nodes * D // L, L)
    )
    out = _hier_softmax_sc(center, context, emb_w, probs_w, num_nodes)
    return out.reshape(-1, 1)
